# baseline (speedup 1.0000x reference)
import functools

import jax
import jax.numpy as jnp
from jax import lax
from jax.experimental import pallas as pl
from jax.experimental.pallas import tpu as pltpu
from jax.experimental.pallas import tpu_sc as plsc

B, S, D, M, V = 64, 50, 128, 2048, 100000
TOPK = 12
NW = 32
ROWS_PER_W = B // NW
LANES = 16
KBW = 64
DC = D // LANES

_GDN = lax.GatherDimensionNumbers(
    offset_dims=(), collapsed_slice_dims=(0,), start_index_map=(0,))


def _ivec_body(dh_ref, h_ref, w1_ref, b1_ref, w2_ref, b2_ref, out_ref):
    x = dh_ref[...]
    h = h_ref[0]
    hb = jnp.broadcast_to(h[:, None, :], (B, S, D))
    cat = jnp.concatenate([hb, x], axis=2).reshape(B * S, 2 * D)
    t = jnp.tanh(jnp.dot(cat, w1_ref[...],
                         preferred_element_type=jnp.float32) + b1_ref[...])
    q = (jnp.dot(t, w2_ref[...],
                 preferred_element_type=jnp.float32) + b2_ref[...])
    q = q.reshape(B, S, D)
    q = q - jnp.max(q, axis=1, keepdims=True)
    e = jnp.exp(q)
    q = e / jnp.sum(e, axis=1, keepdims=True)
    out_ref[...] = jnp.sum(q * x, axis=1)


def _topk_body(l_ref, win_ref, ctx_ref, out_ref):
    ctxm1 = ctx_ref[:, :1] - 1
    l = jnp.concatenate([l_ref[...], win_ref[:, :1]], axis=1)
    rawpos = lax.broadcasted_iota(jnp.int32, (B, KBW + 1), 1)
    pos = jnp.where(rawpos == KBW, ctxm1, rawpos)
    cols = []
    for _ in range(TOPK):
        v = jnp.max(l, axis=1, keepdims=True)
        idx = jnp.min(jnp.where(l == v, pos, M), axis=1, keepdims=True)
        cols.append(idx)
        l = jnp.where(pos == idx, -jnp.inf, l)
    out_ref[...] = jnp.concatenate(cols, axis=1)


def _shuf(v, perm):
    return lax.gather(v, perm[:, None], _GDN, (1,),
                      mode=lax.GatherScatterMode.PROMISE_IN_BOUNDS)


def _hsum(v, lane):
    for sh in (8, 4, 2, 1):
        v = v + _shuf(v, lane ^ sh)
    return v


def _dots_16(rows_v, base, gp16, iv_chunks, lane):
    out = jnp.zeros((LANES,), jnp.float32)
    for s in range(LANES):
        acc = rows_v[base + s, pl.ds(0, LANES)] * iv_chunks[0]
        for d in range(1, DC):
            acc = acc + (rows_v[base + s, pl.ds(d * LANES, LANES)]
                         * iv_chunks[d])
        out = jnp.where(lane == s, _hsum(acc, lane), out)
    return out * gp16


def _masked_sigmoid(dots16, pos16, kb16, ctx16):
    bad = ((pos16 >= kb16) & (pos16 < ctx16 - 1)) | (pos16 >= ctx16)
    xm = jnp.where(bad, jnp.float32(-1e9), dots16)
    return 1.0 / (1.0 + jnp.exp(-xm))


def _sc_logits(c_know, story64, gp64, storyc, gpc, kb_len, ctx_len, i_vec):
    mesh = plsc.VectorSubcoreMesh(core_axis_name="c", subcore_axis_name="s")

    @functools.partial(
        pl.kernel, mesh=mesh,
        out_type=(jax.ShapeDtypeStruct((B, KBW), jnp.float32),
                  jax.ShapeDtypeStruct((B, LANES), jnp.float32)),
        scratch_types=[
            pltpu.VMEM((KBW,), jnp.int32),
            pltpu.VMEM((LANES,), jnp.int32),
            pltpu.VMEM((KBW,), jnp.float32),
            pltpu.VMEM((LANES,), jnp.float32),
            pltpu.VMEM((D,), jnp.float32),
            pltpu.VMEM((KBW, D), jnp.float32),
            pltpu.VMEM((LANES, D), jnp.float32),
            pltpu.VMEM((KBW,), jnp.float32),
            pltpu.VMEM((B, LANES), jnp.int32),
            pltpu.VMEM((B, LANES), jnp.int32),
            pltpu.SemaphoreType.DMA,
            pltpu.SemaphoreType.DMA,
            pltpu.SemaphoreType.DMA,
        ],
    )
    def k(c_hbm, story_hbm, gp_hbm, storyc_hbm, gpc_hbm, kb_hbm, ctx_hbm,
          iv_hbm, out_hbm, win_hbm,
          story_v, storyw_v, gp_v, gpw_v, ivv_v, rows_v, rowsw_v,
          outbuf, kb_v, ctx_v, sem_s, sem_w, sem_r):
        cid = lax.axis_index("c")
        sid = lax.axis_index("s")
        w = sid * 2 + cid
        pltpu.sync_copy(kb_hbm, kb_v)
        pltpu.sync_copy(ctx_hbm, ctx_v)
        lane = lax.iota(jnp.int32, LANES)
        for r in range(ROWS_PER_W):
            b = w * ROWS_PER_W + r
            kb16 = kb_v[b]
            ctx16 = ctx_v[b]

            cp_s = pltpu.async_copy(story_hbm.at[b], story_v, sem_s)
            cp_sw = pltpu.async_copy(storyc_hbm.at[b], storyw_v, sem_w)
            cp_g = pltpu.async_copy(gp_hbm.at[b], gp_v, sem_r)
            cp_gw = pltpu.async_copy(gpc_hbm.at[b], gpw_v, sem_r)
            cp_i = pltpu.async_copy(iv_hbm.at[b], ivv_v, sem_r)

            cp_s.wait()
            cp_rows = pltpu.async_copy(c_hbm.at[story_v], rows_v, sem_r)
            cp_sw.wait()
            cp_roww = pltpu.async_copy(c_hbm.at[storyw_v], rowsw_v, sem_r)

            cp_g.wait()
            cp_gw.wait()
            cp_i.wait()
            cp_rows.wait()
            cp_roww.wait()

            iv_chunks = [ivv_v[pl.ds(d * LANES, LANES)] for d in range(DC)]
            for c in range(KBW // LANES):
                gp16 = gp_v[pl.ds(c * LANES, LANES)]
                dots16 = _dots_16(rows_v, c * LANES, gp16, iv_chunks, lane)
                pos16 = c * LANES + lane
                outbuf[pl.ds(c * LANES, LANES)] = _masked_sigmoid(
                    dots16, pos16, kb16, ctx16)

            gpw16 = gpw_v[pl.ds(0, LANES)]
            dotsw = _dots_16(rowsw_v, 0, gpw16, iv_chunks, lane)
            sigw = _masked_sigmoid(dotsw, ctx16 - 1, kb16, ctx16)

            pltpu.sync_copy(outbuf, out_hbm.at[b])
            gpw_v[...] = sigw
            pltpu.sync_copy(gpw_v, win_hbm.at[b])

    return k(c_know, story64, gp64, storyc, gpc, kb_len, ctx_len, i_vec)


def kernel(dh_outputs, dh_hidden, global_pointer, batch_size, story, domain,
           context_len, kb_len, conv_len, memory_mask, memory_story,
           W1, b1, W2, b2, C_know):
    i_vec = pl.pallas_call(
        _ivec_body,
        out_shape=jax.ShapeDtypeStruct((B, D), jnp.float32),
    )(dh_outputs, dh_hidden, W1, b1.reshape(1, D), W2, b2.reshape(1, D))

    ctx_i = context_len.astype(jnp.int32)
    kb_b = jnp.broadcast_to(kb_len.astype(jnp.int32)[:, None], (B, LANES))
    ctx_b = jnp.broadcast_to(ctx_i[:, None], (B, LANES))
    ctxm1 = (ctx_i - 1)[:, None]
    story64 = story[:, :KBW]
    gp64 = global_pointer[:, :KBW]
    storyc = jnp.broadcast_to(
        jnp.take_along_axis(story, ctxm1, axis=1), (B, LANES))
    gpc = jnp.broadcast_to(
        jnp.take_along_axis(global_pointer, ctxm1, axis=1), (B, LANES))

    logits64, win = _sc_logits(C_know, story64, gp64, storyc, gpc,
                               kb_b, ctx_b, i_vec)

    toppi = pl.pallas_call(
        _topk_body,
        out_shape=jax.ShapeDtypeStruct((B, TOPK), jnp.int32),
    )(logits64, win, ctx_b)
    return toppi, i_vec


# speedup vs baseline: 1.1114x; 1.1114x over previous
import functools

import jax
import jax.numpy as jnp
from jax import lax
from jax.experimental import pallas as pl
from jax.experimental.pallas import tpu as pltpu
from jax.experimental.pallas import tpu_sc as plsc

B, S, D, M, V = 64, 50, 128, 2048, 100000
TOPK = 12
NW = 32
ROWS_PER_W = B // NW
LANES = 16
KBW = 64
DC = D // LANES

_GDN = lax.GatherDimensionNumbers(
    offset_dims=(), collapsed_slice_dims=(0,), start_index_map=(0,))


def _ivec_body(dh_ref, h_ref, w1_ref, b1_ref, w2_ref, b2_ref, out_ref):
    x = dh_ref[...]
    h = h_ref[0]
    hb = jnp.broadcast_to(h[:, None, :], (B, S, D))
    cat = jnp.concatenate([hb, x], axis=2).reshape(B * S, 2 * D)
    t = jnp.tanh(jnp.dot(cat, w1_ref[...],
                         preferred_element_type=jnp.float32) + b1_ref[...])
    q = (jnp.dot(t, w2_ref[...],
                 preferred_element_type=jnp.float32) + b2_ref[...])
    q = q.reshape(B, S, D)
    q = q - jnp.max(q, axis=1, keepdims=True)
    e = jnp.exp(q)
    q = e / jnp.sum(e, axis=1, keepdims=True)
    out_ref[...] = jnp.sum(q * x, axis=1)


def _topk_body(l_ref, ctx_ref, out_ref):
    ctxm1 = ctx_ref[:, :1] - 1
    l = l_ref[...]
    rawpos = lax.broadcasted_iota(jnp.int32, (B, 2 * KBW), 1)
    pos = jnp.where(rawpos == KBW, ctxm1, rawpos)
    cols = []
    for _ in range(TOPK):
        v = jnp.max(l, axis=1, keepdims=True)
        idx = jnp.min(jnp.where(l == v, pos, M), axis=1, keepdims=True)
        cols.append(idx)
        l = jnp.where(pos == idx, -jnp.inf, l)
    out_ref[...] = jnp.concatenate(cols, axis=1)


def _shuf(v, perm):
    return lax.gather(v, perm[:, None], _GDN, (1,),
                      mode=lax.GatherScatterMode.PROMISE_IN_BOUNDS)


def _hsum(v, lane):
    for sh in (8, 4, 2, 1):
        v = v + _shuf(v, lane ^ sh)
    return v


def _dots_16(rows_v, base, gp16, iv_chunks, lane):
    out = jnp.zeros((LANES,), jnp.float32)
    for s in range(LANES):
        acc = rows_v[base + s, pl.ds(0, LANES)] * iv_chunks[0]
        for d in range(1, DC):
            acc = acc + (rows_v[base + s, pl.ds(d * LANES, LANES)]
                         * iv_chunks[d])
        out = jnp.where(lane == s, _hsum(acc, lane), out)
    return out * gp16


def _masked_sigmoid(dots16, pos16, kb16, ctx16):
    bad = ((pos16 >= kb16) & (pos16 < ctx16 - 1)) | (pos16 >= ctx16)
    xm = jnp.where(bad, jnp.float32(-1e9), dots16)
    return 1.0 / (1.0 + jnp.exp(-xm))


def _sc_logits(c_know, story, gp, kb_len, ctx_len, i_vec):
    mesh = plsc.VectorSubcoreMesh(core_axis_name="c", subcore_axis_name="s")

    @functools.partial(
        pl.kernel, mesh=mesh,
        out_type=jax.ShapeDtypeStruct((B, 2 * KBW), jnp.float32),
        scratch_types=[
            pltpu.VMEM((KBW,), jnp.int32),
            pltpu.VMEM((LANES,), jnp.int32),
            pltpu.VMEM((LANES,), jnp.int32),
            pltpu.VMEM((KBW,), jnp.float32),
            pltpu.VMEM((LANES,), jnp.float32),
            pltpu.VMEM((D,), jnp.float32),
            pltpu.VMEM((KBW, D), jnp.float32),
            pltpu.VMEM((LANES, D), jnp.float32),
            pltpu.VMEM((2 * KBW,), jnp.float32),
            pltpu.VMEM((B, LANES), jnp.int32),
            pltpu.VMEM((B, LANES), jnp.int32),
            pltpu.SemaphoreType.DMA,
            pltpu.SemaphoreType.DMA,
            pltpu.SemaphoreType.DMA,
        ],
    )
    def k(c_hbm, story_hbm, gp_hbm, kb_hbm, ctx_hbm, iv_hbm, out_hbm,
          story_v, storyw_v, idxw_v, gp_v, gpw_v, ivv_v, rows_v, rowsw_v,
          outbuf, kb_v, ctx_v, sem_s, sem_w, sem_r):
        cid = lax.axis_index("c")
        sid = lax.axis_index("s")
        w = sid * 2 + cid
        pltpu.sync_copy(kb_hbm, kb_v)
        pltpu.sync_copy(ctx_hbm, ctx_v)
        lane = lax.iota(jnp.int32, LANES)
        zero16 = jnp.zeros((LANES,), jnp.float32)
        for r in range(ROWS_PER_W):
            b = w * ROWS_PER_W + r
            kb16 = kb_v[b]
            ctx16 = ctx_v[b]

            row0 = pl.multiple_of(b * M, LANES)
            rowi = pl.multiple_of(b * D, LANES)
            cp_s = pltpu.async_copy(story_hbm.at[pl.ds(row0, KBW)], story_v, sem_s)
            cp_g = pltpu.async_copy(gp_hbm.at[pl.ds(row0, KBW)], gp_v, sem_r)
            cp_i = pltpu.async_copy(iv_hbm.at[pl.ds(rowi, D)], ivv_v, sem_r)

            idxw_v[...] = b * M + ctx16 - 1
            cp_sw = pltpu.async_copy(story_hbm.at[idxw_v], storyw_v, sem_w)
            cp_gw = pltpu.async_copy(gp_hbm.at[idxw_v], gpw_v, sem_r)

            cp_s.wait()
            cp_rows = pltpu.async_copy(c_hbm.at[story_v], rows_v, sem_r)
            cp_sw.wait()
            cp_roww = pltpu.async_copy(c_hbm.at[storyw_v], rowsw_v, sem_r)

            cp_g.wait()
            cp_gw.wait()
            cp_i.wait()
            cp_rows.wait()
            cp_roww.wait()

            iv_chunks = [ivv_v[pl.ds(d * LANES, LANES)] for d in range(DC)]
            for c in range(KBW // LANES):
                gp16 = gp_v[pl.ds(c * LANES, LANES)]
                dots16 = _dots_16(rows_v, c * LANES, gp16, iv_chunks, lane)
                pos16 = c * LANES + lane
                outbuf[pl.ds(c * LANES, LANES)] = _masked_sigmoid(
                    dots16, pos16, kb16, ctx16)

            gpw16 = gpw_v[pl.ds(0, LANES)]
            dotsw = _dots_16(rowsw_v, 0, gpw16, iv_chunks, lane)
            sigw = _masked_sigmoid(dotsw, ctx16 - 1, kb16, ctx16)

            outbuf[pl.ds(KBW, LANES)] = jnp.where(lane == 0, sigw, 0.0)
            for c in range(KBW // LANES + 1, 2 * KBW // LANES):
                outbuf[pl.ds(c * LANES, LANES)] = zero16
            pltpu.sync_copy(outbuf, out_hbm.at[b])

    return k(c_know, story, gp, kb_len, ctx_len, i_vec)


def kernel(dh_outputs, dh_hidden, global_pointer, batch_size, story, domain,
           context_len, kb_len, conv_len, memory_mask, memory_story,
           W1, b1, W2, b2, C_know):
    i_vec = pl.pallas_call(
        _ivec_body,
        out_shape=jax.ShapeDtypeStruct((B, D), jnp.float32),
    )(dh_outputs, dh_hidden, W1, b1.reshape(1, D), W2, b2.reshape(1, D))

    kb_b = jnp.broadcast_to(kb_len.astype(jnp.int32)[:, None], (B, LANES))
    ctx_b = jnp.broadcast_to(context_len.astype(jnp.int32)[:, None], (B, LANES))

    logits = _sc_logits(C_know, story.reshape(B * M),
                        global_pointer.reshape(B * M),
                        kb_b, ctx_b, i_vec.reshape(B * D))

    toppi = pl.pallas_call(
        _topk_body,
        out_shape=jax.ShapeDtypeStruct((B, TOPK), jnp.int32),
    )(logits, ctx_b)
    return toppi, i_vec


# speedup vs baseline: 1.2643x; 1.1375x over previous
import functools

import jax
import jax.numpy as jnp
from jax import lax
from jax.experimental import pallas as pl
from jax.experimental.pallas import tpu as pltpu
from jax.experimental.pallas import tpu_sc as plsc

B, S, D, M, V = 64, 50, 128, 2048, 100000
TOPK = 12
NW = 32
ROWS_PER_W = B // NW
LANES = 16
KBW = 64


def _ivec_body(dh_ref, h_ref, w1_ref, b1_ref, w2_ref, b2_ref, out_ref):
    x = dh_ref[...]
    h = h_ref[0]
    hb = jnp.broadcast_to(h[:, None, :], (B, S, D))
    cat = jnp.concatenate([hb, x], axis=2).reshape(B * S, 2 * D)
    t = jnp.tanh(jnp.dot(cat, w1_ref[...],
                         preferred_element_type=jnp.float32) + b1_ref[...])
    q = (jnp.dot(t, w2_ref[...],
                 preferred_element_type=jnp.float32) + b2_ref[...])
    q = q.reshape(B, S, D)
    q = q - jnp.max(q, axis=1, keepdims=True)
    e = jnp.exp(q)
    q = e / jnp.sum(e, axis=1, keepdims=True)
    out_ref[...] = jnp.sum(q * x, axis=1)


def _pick_own_batch(R, n):
    R3 = R.reshape(B, n, B)
    bb = lax.broadcasted_iota(jnp.int32, (B, n, B), 0)
    jb = lax.broadcasted_iota(jnp.int32, (B, n, B), 2)
    return jnp.sum(jnp.where(jb == bb, R3, 0.0), axis=2)


def _final_body(rows_ref, roww_ref, gp_ref, kb_ref, ctx_ref, iv_ref, out_ref):
    iv = iv_ref[...]
    gpp = gp_ref[...]
    rowsS = rows_ref[...] * gpp[:, :KBW, None]
    rows2 = rowsS.reshape(B * KBW, D)
    R = lax.dot_general(rows2, iv, (((1,), (1,)), ((), ())))
    out64 = _pick_own_batch(R, KBW)
    rowwS = roww_ref[...] * gpp[:, KBW:KBW + 1, None]
    roww2 = rowwS.reshape(B * LANES, D)
    Rw = lax.dot_general(roww2, iv, (((1,), (1,)), ((), ())))
    outw = _pick_own_batch(Rw, LANES)[:, :1]

    kb = kb_ref[:, :1]
    ctx = ctx_ref[:, :1]
    pos64 = lax.broadcasted_iota(jnp.int32, (B, KBW), 1)
    bad64 = ((pos64 >= kb) & (pos64 < ctx - 1)) | (pos64 >= ctx)
    x64 = jnp.where(bad64, jnp.float32(-1e9), out64)
    sig64 = 1.0 / (1.0 + jnp.exp(-x64))
    win = 1.0 / (1.0 + jnp.exp(-outw))

    l = jnp.concatenate(
        [sig64, win, jnp.zeros((B, KBW - 1), jnp.float32)], axis=1)
    rawpos = lax.broadcasted_iota(jnp.int32, (B, 2 * KBW), 1)
    pos = jnp.where(rawpos == KBW, ctx - 1, rawpos)
    cols = []
    for _ in range(TOPK):
        v = jnp.max(l, axis=1, keepdims=True)
        idx = jnp.min(jnp.where(l == v, pos, M), axis=1, keepdims=True)
        cols.append(idx)
        l = jnp.where(pos == idx, -jnp.inf, l)
    out_ref[...] = jnp.concatenate(cols, axis=1)


def _sc_gather(c_know, story, gp, ctx_len):
    mesh = plsc.VectorSubcoreMesh(core_axis_name="c", subcore_axis_name="s")

    @functools.partial(
        pl.kernel, mesh=mesh,
        out_type=(jax.ShapeDtypeStruct((B, KBW, D), jnp.float32),
                  jax.ShapeDtypeStruct((B, LANES, D), jnp.float32),
                  jax.ShapeDtypeStruct((B, 2 * KBW), jnp.float32)),
        scratch_types=[
            pltpu.VMEM((KBW,), jnp.int32),
            pltpu.VMEM((LANES,), jnp.int32),
            pltpu.VMEM((LANES,), jnp.int32),
            pltpu.VMEM((KBW,), jnp.float32),
            pltpu.VMEM((LANES,), jnp.float32),
            pltpu.VMEM((KBW, D), jnp.float32),
            pltpu.VMEM((LANES, D), jnp.float32),
            pltpu.VMEM((2 * KBW,), jnp.float32),
            pltpu.VMEM((B, LANES), jnp.int32),
            pltpu.SemaphoreType.DMA,
            pltpu.SemaphoreType.DMA,
            pltpu.SemaphoreType.DMA,
        ],
    )
    def k(c_hbm, story_hbm, gp_hbm, ctx_hbm, rows_hbm, roww_hbm, gpp_hbm,
          story_v, storyw_v, idxw_v, gp_v, gpw_v, rows_v, rowsw_v,
          gpbuf, ctx_v, sem_s, sem_w, sem_r):
        cid = lax.axis_index("c")
        sid = lax.axis_index("s")
        w = sid * 2 + cid
        pltpu.sync_copy(ctx_hbm, ctx_v)
        zero16 = jnp.zeros((LANES,), jnp.float32)
        for r in range(ROWS_PER_W):
            b = w * ROWS_PER_W + r
            ctx16 = ctx_v[b]

            row0 = pl.multiple_of(b * M, LANES)
            cp_s = pltpu.async_copy(story_hbm.at[pl.ds(row0, KBW)], story_v, sem_s)
            cp_g = pltpu.async_copy(gp_hbm.at[pl.ds(row0, KBW)], gp_v, sem_r)

            idxw_v[...] = b * M + ctx16 - 1
            cp_sw = pltpu.async_copy(story_hbm.at[idxw_v], storyw_v, sem_w)
            cp_gw = pltpu.async_copy(gp_hbm.at[idxw_v], gpw_v, sem_r)

            cp_s.wait()
            cp_rows = pltpu.async_copy(c_hbm.at[story_v], rows_v, sem_r)
            cp_sw.wait()
            cp_roww = pltpu.async_copy(c_hbm.at[storyw_v], rowsw_v, sem_r)

            cp_g.wait()
            cp_gw.wait()
            for c in range(KBW // LANES):
                gpbuf[pl.ds(c * LANES, LANES)] = gp_v[pl.ds(c * LANES, LANES)]
            gpbuf[pl.ds(KBW, LANES)] = gpw_v[pl.ds(0, LANES)]
            for c in range(KBW // LANES + 1, 2 * KBW // LANES):
                gpbuf[pl.ds(c * LANES, LANES)] = zero16
            pltpu.sync_copy(gpbuf, gpp_hbm.at[b])

            cp_rows.wait()
            cp_roww.wait()
            pltpu.sync_copy(rows_v, rows_hbm.at[b])
            pltpu.sync_copy(rowsw_v, roww_hbm.at[b])

    return k(c_know, story, gp, ctx_len)


def kernel(dh_outputs, dh_hidden, global_pointer, batch_size, story, domain,
           context_len, kb_len, conv_len, memory_mask, memory_story,
           W1, b1, W2, b2, C_know):
    i_vec = pl.pallas_call(
        _ivec_body,
        out_shape=jax.ShapeDtypeStruct((B, D), jnp.float32),
    )(dh_outputs, dh_hidden, W1, b1.reshape(1, D), W2, b2.reshape(1, D))

    kb_b = jnp.broadcast_to(kb_len.astype(jnp.int32)[:, None], (B, LANES))
    ctx_b = jnp.broadcast_to(context_len.astype(jnp.int32)[:, None], (B, LANES))

    rows3, roww3, gpp = _sc_gather(C_know, story.reshape(B * M),
                                   global_pointer.reshape(B * M), ctx_b)

    toppi = pl.pallas_call(
        _final_body,
        out_shape=jax.ShapeDtypeStruct((B, TOPK), jnp.int32),
    )(rows3, roww3, gpp, kb_b, ctx_b, i_vec)
    return toppi, i_vec


# speedup vs baseline: 1.3198x; 1.0439x over previous
import functools

import jax
import jax.numpy as jnp
from jax import lax
from jax.experimental import pallas as pl
from jax.experimental.pallas import tpu as pltpu
from jax.experimental.pallas import tpu_sc as plsc

B, S, D, M, V = 64, 50, 128, 2048, 100000
TOPK = 12
NW = 32
ROWS_PER_W = B // NW
LANES = 16
KBW = 64


def _ivec_body(dh_ref, h_ref, w1_ref, b1_ref, w2_ref, b2_ref, out_ref):
    x = dh_ref[...]
    h = h_ref[0]
    hb = jnp.broadcast_to(h[:, None, :], (B, S, D))
    cat = jnp.concatenate([hb, x], axis=2).reshape(B * S, 2 * D)
    t = jnp.tanh(jnp.dot(cat, w1_ref[...],
                         preferred_element_type=jnp.float32) + b1_ref[...])
    q = (jnp.dot(t, w2_ref[...],
                 preferred_element_type=jnp.float32) + b2_ref[...])
    q = q.reshape(B, S, D)
    q = q - jnp.max(q, axis=1, keepdims=True)
    e = jnp.exp(q)
    q = e / jnp.sum(e, axis=1, keepdims=True)
    out_ref[...] = jnp.sum(q * x, axis=1)


def _pick_own_batch(R, n):
    R3 = R.reshape(B, n, B)
    bb = lax.broadcasted_iota(jnp.int32, (B, n, B), 0)
    jb = lax.broadcasted_iota(jnp.int32, (B, n, B), 2)
    return jnp.sum(jnp.where(jb == bb, R3, 0.0), axis=2)


def _final_body(rows_ref, roww_ref, gp_ref, kb_ref, ctx_ref, iv_ref, out_ref):
    iv = iv_ref[...]
    gpp = gp_ref[...]
    rowsS = rows_ref[...] * gpp[:, :KBW, None]
    rows2 = rowsS.reshape(B * KBW, D)
    R = lax.dot_general(rows2, iv, (((1,), (1,)), ((), ())))
    out64 = _pick_own_batch(R, KBW)
    rowwS = roww_ref[...] * gpp[:, KBW:KBW + 1, None]
    roww2 = rowwS.reshape(B * LANES, D)
    Rw = lax.dot_general(roww2, iv, (((1,), (1,)), ((), ())))
    outw = _pick_own_batch(Rw, LANES)[:, :1]

    kb = kb_ref[:, :1]
    ctx = ctx_ref[:, :1]
    pos64 = lax.broadcasted_iota(jnp.int32, (B, KBW), 1)
    bad64 = ((pos64 >= kb) & (pos64 < ctx - 1)) | (pos64 >= ctx)
    x64 = jnp.where(bad64, jnp.float32(-1e9), out64)
    sig64 = 1.0 / (1.0 + jnp.exp(-x64))
    win = 1.0 / (1.0 + jnp.exp(-outw))

    l = jnp.concatenate(
        [sig64, win, jnp.zeros((B, KBW - 1), jnp.float32)], axis=1)
    rawpos = lax.broadcasted_iota(jnp.int32, (B, 2 * KBW), 1)
    pos = jnp.where(rawpos == KBW, ctx - 1, rawpos)
    cols = []
    for _ in range(TOPK):
        v = jnp.max(l, axis=1, keepdims=True)
        idx = jnp.min(jnp.where(l == v, pos, M), axis=1, keepdims=True)
        cols.append(idx)
        l = jnp.where(pos == idx, -jnp.inf, l)
    out_ref[...] = jnp.concatenate(cols, axis=1)


def _sc_gather(c_know, story, gp, ctx_len):
    mesh = plsc.VectorSubcoreMesh(core_axis_name="c", subcore_axis_name="s")

    @functools.partial(
        pl.kernel, mesh=mesh,
        out_type=(jax.ShapeDtypeStruct((B, KBW, D), jnp.float32),
                  jax.ShapeDtypeStruct((B, LANES, D), jnp.float32),
                  jax.ShapeDtypeStruct((B, 2 * KBW), jnp.float32)),
        scratch_types=(
            [pltpu.VMEM((KBW,), jnp.int32)] * ROWS_PER_W
            + [pltpu.VMEM((LANES,), jnp.int32)] * ROWS_PER_W
            + [pltpu.VMEM((LANES,), jnp.int32)] * ROWS_PER_W
            + [pltpu.VMEM((KBW,), jnp.float32)] * ROWS_PER_W
            + [pltpu.VMEM((LANES,), jnp.float32)] * ROWS_PER_W
            + [pltpu.VMEM((KBW, D), jnp.float32)] * ROWS_PER_W
            + [pltpu.VMEM((LANES, D), jnp.float32)] * ROWS_PER_W
            + [pltpu.VMEM((2 * KBW,), jnp.float32)] * ROWS_PER_W
            + [pltpu.VMEM((B, LANES), jnp.int32)]
            + [pltpu.SemaphoreType.DMA] * (2 * ROWS_PER_W)
            + [pltpu.SemaphoreType.DMA] * ROWS_PER_W
            + [pltpu.SemaphoreType.DMA] * ROWS_PER_W
            + [pltpu.SemaphoreType.DMA]
        ),
    )
    def k(c_hbm, story_hbm, gp_hbm, ctx_hbm, rows_hbm, roww_hbm, gpp_hbm,
          *refs):
        n = ROWS_PER_W
        story_v = refs[0:n]
        storyw_v = refs[n:2 * n]
        idxw_v = refs[2 * n:3 * n]
        gp_v = refs[3 * n:4 * n]
        gpw_v = refs[4 * n:5 * n]
        rows_v = refs[5 * n:6 * n]
        rowsw_v = refs[6 * n:7 * n]
        gpbuf = refs[7 * n:8 * n]
        ctx_v = refs[8 * n]
        sem_s = refs[8 * n + 1:8 * n + 1 + n]
        sem_w = refs[8 * n + 1 + n:8 * n + 1 + 2 * n]
        sem_g = refs[8 * n + 1 + 2 * n:8 * n + 1 + 3 * n]
        sem_r = refs[8 * n + 1 + 3 * n:8 * n + 1 + 4 * n]
        sem_o = refs[8 * n + 1 + 4 * n]

        cid = lax.axis_index("c")
        sid = lax.axis_index("s")
        w = sid * 2 + cid
        pltpu.sync_copy(ctx_hbm, ctx_v)
        zero16 = jnp.zeros((LANES,), jnp.float32)

        cps, cpw, cpg = [], [], []
        for r in range(ROWS_PER_W):
            b = w * ROWS_PER_W + r
            ctx16 = ctx_v[b]
            row0 = pl.multiple_of(b * M, LANES)
            cps.append(pltpu.async_copy(
                story_hbm.at[pl.ds(row0, KBW)], story_v[r], sem_s[r]))
            cpg.append(pltpu.async_copy(
                gp_hbm.at[pl.ds(row0, KBW)], gp_v[r], sem_g[r]))
            idxw_v[r][...] = b * M + ctx16 - 1
            cpw.append(pltpu.async_copy(
                story_hbm.at[idxw_v[r]], storyw_v[r], sem_w[r]))
            cpg.append(pltpu.async_copy(
                gp_hbm.at[idxw_v[r]], gpw_v[r], sem_g[r]))

        cpr, out_cps = [], []
        for r in range(ROWS_PER_W):
            cps[r].wait()
            cpr.append(pltpu.async_copy(c_hbm.at[story_v[r]], rows_v[r], sem_r[r]))
            cpw[r].wait()
            cpr.append(pltpu.async_copy(c_hbm.at[storyw_v[r]], rowsw_v[r], sem_r[r]))

        for r in range(ROWS_PER_W):
            b = w * ROWS_PER_W + r
            cpg[2 * r].wait()
            cpg[2 * r + 1].wait()
            for c in range(KBW // LANES):
                gpbuf[r][pl.ds(c * LANES, LANES)] = gp_v[r][pl.ds(c * LANES, LANES)]
            gpbuf[r][pl.ds(KBW, LANES)] = gpw_v[r][pl.ds(0, LANES)]
            for c in range(KBW // LANES + 1, 2 * KBW // LANES):
                gpbuf[r][pl.ds(c * LANES, LANES)] = zero16
            out_cps.append(pltpu.async_copy(gpbuf[r], gpp_hbm.at[b], sem_o))

        for r in range(ROWS_PER_W):
            b = w * ROWS_PER_W + r
            cpr[2 * r].wait()
            cpr[2 * r + 1].wait()
            out_cps.append(pltpu.async_copy(rows_v[r], rows_hbm.at[b], sem_o))
            out_cps.append(pltpu.async_copy(rowsw_v[r], roww_hbm.at[b], sem_o))

        for cp in out_cps:
            cp.wait()

    return k(c_know, story, gp, ctx_len)


def kernel(dh_outputs, dh_hidden, global_pointer, batch_size, story, domain,
           context_len, kb_len, conv_len, memory_mask, memory_story,
           W1, b1, W2, b2, C_know):
    i_vec = pl.pallas_call(
        _ivec_body,
        out_shape=jax.ShapeDtypeStruct((B, D), jnp.float32),
    )(dh_outputs, dh_hidden, W1, b1.reshape(1, D), W2, b2.reshape(1, D))

    kb_b = jnp.broadcast_to(kb_len.astype(jnp.int32)[:, None], (B, LANES))
    ctx_b = jnp.broadcast_to(context_len.astype(jnp.int32)[:, None], (B, LANES))

    rows3, roww3, gpp = _sc_gather(C_know, story.reshape(B * M),
                                   global_pointer.reshape(B * M), ctx_b)

    toppi = pl.pallas_call(
        _final_body,
        out_shape=jax.ShapeDtypeStruct((B, TOPK), jnp.int32),
    )(rows3, roww3, gpp, kb_b, ctx_b, i_vec)
    return toppi, i_vec
